# Initial kernel scaffold; baseline (speedup 1.0000x reference)
import functools

import jax
import jax.numpy as jnp
from jax import lax
from jax.experimental import pallas as pl
from jax.experimental.pallas import tpu as pltpu
from jax.experimental.pallas import tpu_sc as plsc

N_NODES = 10000
N_EDGES = 320000
D = 128

NC = 2
NS = 16
NW = NC * NS

CHUNK = 128
E_PAD = 327680
E_PER_TILE = E_PAD // NW
CHUNKS_PER_TILE = E_PER_TILE // CHUNK
N_PAD = 10240
ROWS_PER_TILE = N_PAD // NW

_mesh = plsc.VectorSubcoreMesh(
    core_axis_name="c", subcore_axis_name="s", num_cores=NC, num_subcores=NS
)


@functools.partial(
    pl.kernel,
    out_type=jax.ShapeDtypeStruct((NC, N_PAD, D), jnp.float32),
    mesh=_mesh,
    scratch_types=[
        pltpu.VMEM_SHARED((N_PAD, D), jnp.float32),
        pltpu.VMEM((CHUNKS_PER_TILE, CHUNK), jnp.int32),
        pltpu.VMEM((CHUNKS_PER_TILE, CHUNK), jnp.int32),
        pltpu.VMEM((CHUNK, D), jnp.float32),
        pltpu.SemaphoreType.DMA,
    ],
)
def _sc_aggregate(x_hbm, src_hbm, dst_hbm, out_hbm, acc, src_v, dst_v, rows_v, sem):
    cid = lax.axis_index("c")
    sid = lax.axis_index("s")
    wid = sid * NC + cid

    def _zero_row(r, _):
        for j in range(D // 16):
            rows_v[r, pl.ds(j * 16, 16)] = jnp.zeros((16,), jnp.float32)
        return 0

    lax.fori_loop(0, CHUNK, _zero_row, 0)

    base_rows = wid * ROWS_PER_TILE
    for r0 in range(0, ROWS_PER_TILE, CHUNK):
        n = min(CHUNK, ROWS_PER_TILE - r0)
        pltpu.sync_copy(rows_v.at[pl.ds(0, n)], acc.at[pl.ds(base_rows + r0, n)])

    base_chunk = wid * CHUNKS_PER_TILE
    pltpu.sync_copy(src_hbm.at[pl.ds(base_chunk, CHUNKS_PER_TILE)], src_v)
    pltpu.sync_copy(dst_hbm.at[pl.ds(base_chunk, CHUNKS_PER_TILE)], dst_v)

    plsc.subcore_barrier()

    def _edge_chunk(c, _):
        pltpu.async_copy(x_hbm.at[src_v.at[c]], rows_v, sem).wait()
        pltpu.sync_copy(rows_v, acc.at[dst_v.at[c]], add=True)
        return 0

    lax.fori_loop(0, CHUNKS_PER_TILE, _edge_chunk, 0)

    plsc.subcore_barrier()

    pltpu.sync_copy(
        acc.at[pl.ds(base_rows, ROWS_PER_TILE)],
        out_hbm.at[cid, pl.ds(base_rows, ROWS_PER_TILE)],
    )


_BLK = 1024


def _combine_matmul_body(p_ref, w_ref, b_ref, o_ref):
    p = p_ref[0] + p_ref[1]
    o_ref[...] = (
        jnp.dot(p, w_ref[...], preferred_element_type=jnp.float32) + b_ref[...]
    )


def _combine_matmul(partials, W, b2d):
    return pl.pallas_call(
        _combine_matmul_body,
        grid=(N_PAD // _BLK,),
        in_specs=[
            pl.BlockSpec((NC, _BLK, D), lambda i: (0, i, 0)),
            pl.BlockSpec((D, D), lambda i: (0, 0)),
            pl.BlockSpec((1, D), lambda i: (0, 0)),
        ],
        out_specs=pl.BlockSpec((_BLK, D), lambda i: (i, 0)),
        out_shape=jax.ShapeDtypeStruct((N_PAD, D), jnp.float32),
    )(partials, W, b2d)


def kernel(x, edge_index, W, b):
    src = edge_index[0].astype(jnp.int32)
    dst = edge_index[1].astype(jnp.int32)

    pad = E_PAD - N_EDGES
    pad_idx = jnp.full((pad,), N_NODES, dtype=jnp.int32)
    src2d = jnp.concatenate([src, pad_idx]).reshape(E_PAD // CHUNK, CHUNK)
    dst2d = jnp.concatenate([dst, pad_idx]).reshape(E_PAD // CHUNK, CHUNK)

    x_pad = jnp.zeros((N_PAD, D), jnp.float32).at[:N_NODES].set(x)

    partials = _sc_aggregate(x_pad, src2d, dst2d)
    out = _combine_matmul(partials, W, b.reshape(1, D))
    return (out[:N_NODES],)


# speedup vs baseline: 3.2267x; 3.2267x over previous
import functools

import jax
import jax.numpy as jnp
from jax import lax
from jax.experimental import pallas as pl
from jax.experimental.pallas import tpu as pltpu
from jax.experimental.pallas import tpu_sc as plsc

N_NODES = 10000
N_EDGES = 320000
D = 128

NC = 2
NS = 16
NW = NC * NS

CHUNK = 128
E_PAD = 327680
E_PER_TILE = E_PAD // NW
CHUNKS_PER_TILE = E_PER_TILE // CHUNK
N_PAD = 10240
ROWS_PER_TILE = N_PAD // NS

_mesh = plsc.VectorSubcoreMesh(
    core_axis_name="c", subcore_axis_name="s", num_cores=NC, num_subcores=NS
)


@functools.partial(
    pl.kernel,
    out_type=jax.ShapeDtypeStruct((NC, N_PAD, D), jnp.float32),
    mesh=_mesh,
    scratch_types=[
        pltpu.VMEM_SHARED((N_PAD, D), jnp.float32),
        pltpu.VMEM((CHUNKS_PER_TILE, CHUNK), jnp.int32),
        pltpu.VMEM((CHUNKS_PER_TILE, CHUNK), jnp.int32),
        pltpu.VMEM((CHUNK, D), jnp.float32),
        pltpu.SemaphoreType.DMA,
    ],
)
def _sc_aggregate(x_hbm, src_hbm, dst_hbm, out_hbm, acc, src_v, dst_v, rows_v, sem):
    cid = lax.axis_index("c")
    sid = lax.axis_index("s")
    wid = sid * NC + cid

    def _zero_row(r, _):
        for j in range(D // 16):
            rows_v[r, pl.ds(j * 16, 16)] = jnp.zeros((16,), jnp.float32)
        return 0

    lax.fori_loop(0, CHUNK, _zero_row, 0)

    base_rows = sid * ROWS_PER_TILE
    for r0 in range(0, ROWS_PER_TILE, CHUNK):
        n = min(CHUNK, ROWS_PER_TILE - r0)
        pltpu.sync_copy(rows_v.at[pl.ds(0, n)], acc.at[pl.ds(base_rows + r0, n)])

    base_chunk = wid * CHUNKS_PER_TILE
    pltpu.sync_copy(src_hbm.at[pl.ds(base_chunk, CHUNKS_PER_TILE)], src_v)
    pltpu.sync_copy(dst_hbm.at[pl.ds(base_chunk, CHUNKS_PER_TILE)], dst_v)

    plsc.subcore_barrier()

    def _edge_chunk(c, _):
        pltpu.async_copy(x_hbm.at[src_v.at[c]], rows_v, sem).wait()
        pltpu.sync_copy(rows_v, acc.at[dst_v.at[c]], add=True)
        return 0

    lax.fori_loop(0, CHUNKS_PER_TILE, _edge_chunk, 0)

    plsc.subcore_barrier()

    pltpu.sync_copy(
        acc.at[pl.ds(base_rows, ROWS_PER_TILE)],
        out_hbm.at[cid, pl.ds(base_rows, ROWS_PER_TILE)],
    )


_BLK = 1024


def _combine_matmul_body(p_ref, w_ref, b_ref, o_ref):
    p = p_ref[0] + p_ref[1]
    o_ref[...] = (
        jnp.dot(p, w_ref[...], preferred_element_type=jnp.float32) + b_ref[...]
    )


def _combine_matmul(partials, W, b2d):
    return pl.pallas_call(
        _combine_matmul_body,
        grid=(N_PAD // _BLK,),
        in_specs=[
            pl.BlockSpec((NC, _BLK, D), lambda i: (0, i, 0)),
            pl.BlockSpec((D, D), lambda i: (0, 0)),
            pl.BlockSpec((1, D), lambda i: (0, 0)),
        ],
        out_specs=pl.BlockSpec((_BLK, D), lambda i: (i, 0)),
        out_shape=jax.ShapeDtypeStruct((N_PAD, D), jnp.float32),
    )(partials, W, b2d)


def kernel(x, edge_index, W, b):
    src = edge_index[0].astype(jnp.int32)
    dst = edge_index[1].astype(jnp.int32)

    pad = E_PAD - N_EDGES
    pad_idx = jnp.full((pad,), N_NODES, dtype=jnp.int32)
    src2d = jnp.concatenate([src, pad_idx]).reshape(E_PAD // CHUNK, CHUNK)
    dst2d = jnp.concatenate([dst, pad_idx]).reshape(E_PAD // CHUNK, CHUNK)

    x_pad = jnp.zeros((N_PAD, D), jnp.float32).at[:N_NODES].set(x)

    partials = _sc_aggregate(x_pad, src2d, dst2d)
    out = _combine_matmul(partials, W, b.reshape(1, D))
    return (out[:N_NODES],)


# speedup vs baseline: 3.5318x; 1.0946x over previous
import functools

import jax
import jax.numpy as jnp
from jax import lax
from jax.experimental import pallas as pl
from jax.experimental.pallas import tpu as pltpu
from jax.experimental.pallas import tpu_sc as plsc

N_NODES = 10000
N_EDGES = 320000
D = 128

NC = 2
NS = 16
NW = NC * NS

CHUNK = 128
NBUF = 2
IDX_HALVES = 2
E_PAD = 327680
E_PER_TILE = E_PAD // NW
CHUNKS_PER_TILE = E_PER_TILE // CHUNK
N_PAD = 10240
ROWS_PER_TILE = N_PAD // NS
IDX_HALF = CHUNKS_PER_TILE // IDX_HALVES

_mesh = plsc.VectorSubcoreMesh(
    core_axis_name="c", subcore_axis_name="s", num_cores=NC, num_subcores=NS
)


@functools.partial(
    pl.kernel,
    out_type=jax.ShapeDtypeStruct((NC, N_PAD, D), jnp.float32),
    mesh=_mesh,
    scratch_types=[
        pltpu.VMEM_SHARED((N_PAD, D), jnp.float32),
        pltpu.VMEM((IDX_HALF, CHUNK), jnp.int32),
        pltpu.VMEM((IDX_HALF, CHUNK), jnp.int32),
        pltpu.VMEM((NBUF, CHUNK, D), jnp.float32),
        [pltpu.SemaphoreType.DMA] * NBUF,
    ],
)
def _sc_aggregate(x_hbm, src_hbm, dst_hbm, out_hbm, acc, src_v, dst_v, rows_v, sems):
    cid = lax.axis_index("c")
    sid = lax.axis_index("s")
    wid = sid * NC + cid

    def _zero_row(r, _):
        for j in range(D // 16):
            rows_v[0, r, pl.ds(j * 16, 16)] = jnp.zeros((16,), jnp.float32)
        return 0

    lax.fori_loop(0, CHUNK, _zero_row, 0)

    base_rows = sid * ROWS_PER_TILE
    for r0 in range(0, ROWS_PER_TILE, CHUNK):
        n = min(CHUNK, ROWS_PER_TILE - r0)
        pltpu.sync_copy(rows_v.at[0, pl.ds(0, n)], acc.at[pl.ds(base_rows + r0, n)])

    plsc.subcore_barrier()

    base_chunk = wid * CHUNKS_PER_TILE
    for h in range(IDX_HALVES):
        hbase = base_chunk + h * IDX_HALF
        pltpu.sync_copy(src_hbm.at[pl.ds(hbase, IDX_HALF)], src_v)
        pltpu.sync_copy(dst_hbm.at[pl.ds(hbase, IDX_HALF)], dst_v)

        for b in range(NBUF):
            pltpu.async_copy(x_hbm.at[src_v.at[b]], rows_v.at[b], sems[b])

        def _edge_group(g, _):
            c0 = g * NBUF
            for b in range(NBUF):
                c = c0 + b
                pltpu.make_async_copy(
                    x_hbm.at[src_v.at[c]], rows_v.at[b], sems[b]
                ).wait()
                pltpu.sync_copy(rows_v.at[b], acc.at[dst_v.at[c]], add=True)
                nxt = c + NBUF

                @pl.when(nxt < IDX_HALF)
                def _():
                    pltpu.async_copy(x_hbm.at[src_v.at[nxt]], rows_v.at[b], sems[b])

            return 0

        lax.fori_loop(0, IDX_HALF // NBUF, _edge_group, 0)

    plsc.subcore_barrier()

    pltpu.sync_copy(
        acc.at[pl.ds(base_rows, ROWS_PER_TILE)],
        out_hbm.at[cid, pl.ds(base_rows, ROWS_PER_TILE)],
    )


_BLK = 1024


def _combine_matmul_body(p_ref, w_ref, b_ref, o_ref):
    p = p_ref[0] + p_ref[1]
    o_ref[...] = (
        jnp.dot(p, w_ref[...], preferred_element_type=jnp.float32) + b_ref[...]
    )


def _combine_matmul(partials, W, b2d):
    return pl.pallas_call(
        _combine_matmul_body,
        grid=(N_PAD // _BLK,),
        in_specs=[
            pl.BlockSpec((NC, _BLK, D), lambda i: (0, i, 0)),
            pl.BlockSpec((D, D), lambda i: (0, 0)),
            pl.BlockSpec((1, D), lambda i: (0, 0)),
        ],
        out_specs=pl.BlockSpec((_BLK, D), lambda i: (i, 0)),
        out_shape=jax.ShapeDtypeStruct((N_PAD, D), jnp.float32),
    )(partials, W, b2d)


def kernel(x, edge_index, W, b):
    src = edge_index[0].astype(jnp.int32)
    dst = edge_index[1].astype(jnp.int32)

    pad = E_PAD - N_EDGES
    pad_idx = jnp.full((pad,), N_NODES, dtype=jnp.int32)
    src2d = jnp.concatenate([src, pad_idx]).reshape(E_PAD // CHUNK, CHUNK)
    dst2d = jnp.concatenate([dst, pad_idx]).reshape(E_PAD // CHUNK, CHUNK)

    x_pad = jnp.zeros((N_PAD, D), jnp.float32).at[:N_NODES].set(x)

    partials = _sc_aggregate(x_pad, src2d, dst2d)
    out = _combine_matmul(partials, W, b.reshape(1, D))
    return (out[:N_NODES],)


# speedup vs baseline: 8.8283x; 2.4997x over previous
import functools

import jax
import jax.numpy as jnp
from jax import lax
from jax.experimental import pallas as pl
from jax.experimental.pallas import tpu as pltpu
from jax.experimental.pallas import tpu_sc as plsc

N_NODES = 10000
N_EDGES = 320000
D = 128
DH = D // 2

NC = 2
NS = 16
NW = NC * NS

CHUNK = 64
NBUF = 4
E_PAD = 327680
E_PER_TILE = E_PAD // NS
CHUNKS_PER_TILE = E_PER_TILE // CHUNK
IDX_PARTS = 8
IDX_PART = CHUNKS_PER_TILE // IDX_PARTS
N_PAD = 10240
ROWS_PER_TILE = N_PAD // NS

_mesh = plsc.VectorSubcoreMesh(
    core_axis_name="c", subcore_axis_name="s", num_cores=NC, num_subcores=NS
)


@functools.partial(
    pl.kernel,
    out_type=jax.ShapeDtypeStruct((NC, N_PAD, DH), jnp.float32),
    mesh=_mesh,
    scratch_types=[
        pltpu.VMEM_SHARED((N_PAD, DH), jnp.float32),
        pltpu.VMEM_SHARED((N_PAD, DH), jnp.float32),
        pltpu.VMEM((IDX_PART, CHUNK), jnp.int32),
        pltpu.VMEM((IDX_PART, CHUNK), jnp.int32),
        pltpu.VMEM((NBUF, CHUNK, DH), jnp.float32),
        [pltpu.SemaphoreType.DMA] * NBUF,
    ],
)
def _sc_aggregate(x_hbm, src_hbm, dst_hbm, out_hbm, acc, x_sp, src_v, dst_v,
                  rows_v, sems):
    cid = lax.axis_index("c")
    sid = lax.axis_index("s")

    base_rows = sid * ROWS_PER_TILE
    pltpu.sync_copy(
        x_hbm.at[cid, pl.ds(base_rows, ROWS_PER_TILE)],
        x_sp.at[pl.ds(base_rows, ROWS_PER_TILE)],
    )

    def _zero_row(r, _):
        for j in range(DH // 16):
            rows_v[0, r, pl.ds(j * 16, 16)] = jnp.zeros((16,), jnp.float32)
        return 0

    lax.fori_loop(0, CHUNK, _zero_row, 0)

    for r0 in range(0, ROWS_PER_TILE, CHUNK):
        pltpu.sync_copy(
            rows_v.at[0, pl.ds(0, CHUNK)], acc.at[pl.ds(base_rows + r0, CHUNK)]
        )

    plsc.subcore_barrier()

    base_chunk = sid * CHUNKS_PER_TILE
    for h in range(IDX_PARTS):
        hbase = base_chunk + h * IDX_PART
        pltpu.sync_copy(src_hbm.at[pl.ds(hbase, IDX_PART)], src_v)
        pltpu.sync_copy(dst_hbm.at[pl.ds(hbase, IDX_PART)], dst_v)

        for b in range(NBUF):
            pltpu.async_copy(x_sp.at[src_v.at[b]], rows_v.at[b], sems[b])

        def _edge_group(g, _):
            c0 = g * NBUF
            for b in range(NBUF):
                c = c0 + b
                pltpu.make_async_copy(
                    x_sp.at[src_v.at[c]], rows_v.at[b], sems[b]
                ).wait()
                pltpu.sync_copy(rows_v.at[b], acc.at[dst_v.at[c]], add=True)
                nxt = c + NBUF

                @pl.when(nxt < IDX_PART)
                def _():
                    pltpu.async_copy(x_sp.at[src_v.at[nxt]], rows_v.at[b], sems[b])

            return 0

        lax.fori_loop(0, IDX_PART // NBUF, _edge_group, 0)

    plsc.subcore_barrier()

    pltpu.sync_copy(
        acc.at[pl.ds(base_rows, ROWS_PER_TILE)],
        out_hbm.at[cid, pl.ds(base_rows, ROWS_PER_TILE)],
    )


_BLK = 1024


def _combine_matmul_body(p_ref, w_ref, b_ref, o_ref):
    p = jnp.concatenate([p_ref[0], p_ref[1]], axis=-1)
    o_ref[...] = (
        jnp.dot(p, w_ref[...], preferred_element_type=jnp.float32) + b_ref[...]
    )


def _combine_matmul(partials, W, b2d):
    return pl.pallas_call(
        _combine_matmul_body,
        grid=(N_PAD // _BLK,),
        in_specs=[
            pl.BlockSpec((NC, _BLK, DH), lambda i: (0, i, 0)),
            pl.BlockSpec((D, D), lambda i: (0, 0)),
            pl.BlockSpec((1, D), lambda i: (0, 0)),
        ],
        out_specs=pl.BlockSpec((_BLK, D), lambda i: (i, 0)),
        out_shape=jax.ShapeDtypeStruct((N_PAD, D), jnp.float32),
    )(partials, W, b2d)


def kernel(x, edge_index, W, b):
    src = edge_index[0].astype(jnp.int32)
    dst = edge_index[1].astype(jnp.int32)

    pad = E_PAD - N_EDGES
    pad_idx = jnp.full((pad,), N_NODES, dtype=jnp.int32)
    src2d = jnp.concatenate([src, pad_idx]).reshape(E_PAD // CHUNK, CHUNK)
    dst2d = jnp.concatenate([dst, pad_idx]).reshape(E_PAD // CHUNK, CHUNK)

    x_pad = jnp.zeros((N_PAD, D), jnp.float32).at[:N_NODES].set(x)
    x_split = jnp.stack([x_pad[:, :DH], x_pad[:, DH:]])

    partials = _sc_aggregate(x_split, src2d, dst2d)
    out = _combine_matmul(partials, W, b.reshape(1, D))
    return (out[:N_NODES],)
